# baseline (speedup 1.0000x reference)
import dataclasses
import functools

import jax
import jax.numpy as jnp
from jax import lax
from jax.experimental import pallas as pl
from jax.experimental.pallas import tpu as pltpu
from jax.experimental.pallas import tpu_sc as plsc

N_NODES = 10000
N_EDGES = 320000
D = 128
G = 64

NC = 2
NS = 16
NW = NC * NS
EPT = N_EDGES // NW
K = 96
CHUNKS = 105
EPS = CHUNKS * K
PAD = EPS - EPT
NP = 10240
RPT = NP // NS
TRASH = 10200

_mesh = plsc.VectorSubcoreMesh(core_axis_name="c", subcore_axis_name="s")


_cp = pltpu.CompilerParams()
if "needs_layout_passes" in pltpu.CompilerParams.__dataclass_fields__:
    _cp = dataclasses.replace(_cp, needs_layout_passes=False)


@functools.partial(
    pl.kernel,
    mesh=_mesh,
    compiler_params=_cp,
    out_type=jax.ShapeDtypeStruct((NW * NP,), jnp.float32),
    scratch_types=[
        pltpu.VMEM((EPT,), jnp.int32),
        pltpu.VMEM((NP,), jnp.float32),
    ],
)
def _sc_degree(dst_hbm, zeros_hbm, out_hbm, dsts_v, cnt_v):
    cid = lax.axis_index("c")
    sid = lax.axis_index("s")
    wid = sid * NC + cid
    base = pl.multiple_of(wid * EPT, 8)
    pltpu.sync_copy(dst_hbm.at[pl.ds(base, EPT)], dsts_v)
    pltpu.sync_copy(zeros_hbm, cnt_v)
    ones = jnp.ones((16,), jnp.float32)

    def body(j, c):
        idx = dsts_v[pl.ds(j * 16, 16)]
        plsc.addupdate_scatter(cnt_v, [idx], ones)
        return c

    lax.fori_loop(0, EPT // 16, body, 0)
    out0 = pl.multiple_of(wid * NP, 8)
    pltpu.sync_copy(cnt_v, out_hbm.at[pl.ds(out0, NP)])


@functools.partial(
    pl.kernel,
    mesh=_mesh,
    out_type=jax.ShapeDtypeStruct((NC * NP, D), jnp.float32),
    scratch_types=[
        pltpu.VMEM((EPS,), jnp.int32),
        pltpu.VMEM((CHUNKS, K), jnp.int32),
        pltpu.VMEM((K, D), jnp.float32),
        pltpu.VMEM((K, D), jnp.float32),
        pltpu.VMEM_SHARED((NP, D), jnp.float32),
        pltpu.SemaphoreType.DMA,
        pltpu.SemaphoreType.DMA,
    ],
)
def _sc_scatter(hp_hbm, src_hbm, dst_hbm, zeros_hbm, out_hbm,
                srcs_v, dsts_v, rows0_v, rows1_v, acc_sh, sem0, sem1):
    cid = lax.axis_index("c")
    sid = lax.axis_index("s")
    wid = sid * NC + cid
    row0 = pl.multiple_of(sid * RPT, 8)
    base = pl.multiple_of(wid * EPS, 8)
    pltpu.sync_copy(src_hbm.at[pl.ds(base, EPS)], srcs_v)
    pltpu.sync_copy(dst_hbm.at[wid], dsts_v)
    pltpu.sync_copy(zeros_hbm, acc_sh.at[pl.ds(row0, RPT)])
    plsc.subcore_barrier()

    def src_at(i):
        return srcs_v.at[pl.ds(pl.multiple_of(i * K, 8), K)]

    pltpu.async_copy(hp_hbm.at[src_at(0)], rows0_v, sem0)

    def body(q, c):
        i0 = 2 * q
        i1 = i0 + 1
        i2 = i0 + 2
        pltpu.async_copy(hp_hbm.at[src_at(i1)], rows1_v, sem1)
        pltpu.make_async_copy(hp_hbm.at[src_at(i0)], rows0_v, sem0).wait()
        pltpu.sync_copy(rows0_v, acc_sh.at[dsts_v.at[i0]], add=True)
        pltpu.async_copy(hp_hbm.at[src_at(i2)], rows0_v, sem0)
        pltpu.make_async_copy(hp_hbm.at[src_at(i1)], rows1_v, sem1).wait()
        pltpu.sync_copy(rows1_v, acc_sh.at[dsts_v.at[i1]], add=True)
        return c

    lax.fori_loop(0, (CHUNKS - 1) // 2, body, 0)
    last = CHUNKS - 1
    pltpu.make_async_copy(hp_hbm.at[src_at(last)], rows0_v, sem0).wait()
    pltpu.sync_copy(rows0_v, acc_sh.at[dsts_v.at[last]], add=True)
    plsc.subcore_barrier()
    out_row0 = pl.multiple_of(cid * NP + sid * RPT, 8)
    pltpu.sync_copy(acc_sh.at[pl.ds(row0, RPT)], out_hbm.at[pl.ds(out_row0, RPT)])


def _dis_from_parts(parts_ref):
    ones = jnp.ones((NW, 1), jnp.float32)
    deg = lax.dot_general(parts_ref[...], ones, (((0,), (0,)), ((), ())),
                          preferred_element_type=jnp.float32)
    return lax.rsqrt(deg[0:N_NODES] + 1.0)


def _tc_prescale(x_ref, w_ref, parts_ref, out_ref):
    dis = _dis_from_parts(parts_ref)
    h = jnp.dot(x_ref[...], w_ref[...], preferred_element_type=jnp.float32)
    out_ref[...] = h * dis


def _tc_mid(s_ref, hp_ref, parts_ref, b_ref, w_ref, out_ref):
    dis = _dis_from_parts(parts_ref)
    s = s_ref[0:N_NODES] + s_ref[NP:NP + N_NODES] + hp_ref[...]
    h = jnp.maximum(dis * s + b_ref[...], 0.0)
    out_ref[...] = jnp.dot(h, w_ref[...], preferred_element_type=jnp.float32) * dis


def _tc_head(s_ref, hp_ref, parts_ref, b_ref, batch_ref, wl_ref, bl_ref, out_ref):
    dis = _dis_from_parts(parts_ref)
    s = s_ref[0:N_NODES] + s_ref[NP:NP + N_NODES] + hp_ref[...]
    h = jnp.maximum(dis * s + b_ref[...], 0.0)
    gid = lax.broadcasted_iota(jnp.int32, (G, N_NODES), 0)
    p = (gid == batch_ref[...]).astype(jnp.float32)
    sums = jnp.dot(p, h, preferred_element_type=jnp.float32)
    counts = jnp.sum(p, axis=1, keepdims=True)
    g = sums / jnp.maximum(counts, 1.0)
    logit = jnp.dot(g, wl_ref[...], preferred_element_type=jnp.float32) + bl_ref[...]
    out_ref[...] = 1.0 / (1.0 + jnp.exp(-logit))


def kernel(x, edge_index, batch, W1, b1, W2, b2, Wl, bl):
    src = edge_index[0].astype(jnp.int32)
    dst = edge_index[1].astype(jnp.int32)
    srcp = jnp.concatenate(
        [src.reshape(NW, EPT), jnp.zeros((NW, PAD), jnp.int32)], axis=1
    ).reshape(-1)
    dst3d = jnp.concatenate(
        [dst.reshape(NW, EPT), jnp.full((NW, PAD), TRASH, jnp.int32)], axis=1
    ).reshape(NW, CHUNKS, K)
    batch2d = batch.astype(jnp.int32).reshape(1, N_NODES)
    zeros_cnt = jnp.zeros((NP,), jnp.float32)
    zeros_feat = jnp.zeros((RPT, D), jnp.float32)
    f32 = jnp.float32

    parts = _sc_degree(dst, zeros_cnt).reshape(NW, NP)

    h1p = pl.pallas_call(
        _tc_prescale, out_shape=jax.ShapeDtypeStruct((N_NODES, D), f32),
    )(x, W1, parts)

    s1 = _sc_scatter(h1p, srcp, dst3d, zeros_feat)

    h2p = pl.pallas_call(
        _tc_mid, out_shape=jax.ShapeDtypeStruct((N_NODES, D), f32),
    )(s1, h1p, parts, b1.reshape(1, D), W2)

    s2 = _sc_scatter(h2p, srcp, dst3d, zeros_feat)

    out = pl.pallas_call(
        _tc_head, out_shape=jax.ShapeDtypeStruct((G, 1), f32),
    )(s2, h2p, parts, b2.reshape(1, D), batch2d, Wl, bl.reshape(1, 1))
    return out.reshape(-1)


# speedup vs baseline: 1.5492x; 1.5492x over previous
import dataclasses
import functools

import jax
import jax.numpy as jnp
from jax import lax
from jax.experimental import pallas as pl
from jax.experimental.pallas import tpu as pltpu
from jax.experimental.pallas import tpu_sc as plsc

N_NODES = 10000
N_EDGES = 320000
D = 128
G = 64

NC = 2
NS = 16
NW = NC * NS
EPT = N_EDGES // NW
K = 80
CHUNKS = EPT // K
NP = 10240
RPT = NP // NS

_mesh = plsc.VectorSubcoreMesh(core_axis_name="c", subcore_axis_name="s")


_cp = pltpu.CompilerParams()
if "needs_layout_passes" in pltpu.CompilerParams.__dataclass_fields__:
    _cp = dataclasses.replace(_cp, needs_layout_passes=False)


@functools.partial(
    pl.kernel,
    mesh=_mesh,
    compiler_params=_cp,
    out_type=jax.ShapeDtypeStruct((NW * NP,), jnp.float32),
    scratch_types=[
        pltpu.VMEM((EPT,), jnp.int32),
        pltpu.VMEM((NP,), jnp.float32),
    ],
)
def _sc_degree(dst_hbm, zeros_hbm, out_hbm, dsts_v, cnt_v):
    cid = lax.axis_index("c")
    sid = lax.axis_index("s")
    wid = sid * NC + cid
    base = pl.multiple_of(wid * EPT, 8)
    pltpu.sync_copy(dst_hbm.at[pl.ds(base, EPT)], dsts_v)
    pltpu.sync_copy(zeros_hbm, cnt_v)
    ones = jnp.ones((16,), jnp.float32)

    def body(j, c):
        idx = dsts_v[pl.ds(j * 16, 16)]
        plsc.addupdate_scatter(cnt_v, [idx], ones)
        return c

    lax.fori_loop(0, EPT // 16, body, 0)
    out0 = pl.multiple_of(wid * NP, 8)
    pltpu.sync_copy(cnt_v, out_hbm.at[pl.ds(out0, NP)])


@functools.partial(
    pl.kernel,
    mesh=_mesh,
    out_type=jax.ShapeDtypeStruct((NC * NP, D), jnp.float32),
    scratch_types=[
        pltpu.VMEM((EPT,), jnp.int32),
        pltpu.VMEM((CHUNKS, K), jnp.int32),
        pltpu.VMEM((K, D), jnp.float32),
        pltpu.VMEM((K, D), jnp.float32),
        pltpu.VMEM_SHARED((NP, D), jnp.float32),
        pltpu.SemaphoreType.DMA,
        pltpu.SemaphoreType.DMA,
    ],
)
def _sc_scatter(hp_hbm, src_hbm, dst_hbm, zeros_hbm, out_hbm,
                srcs_v, dsts_v, rows0_v, rows1_v, acc_sh, sem0, sem1):
    cid = lax.axis_index("c")
    sid = lax.axis_index("s")
    wid = sid * NC + cid
    row0 = pl.multiple_of(sid * RPT, 8)
    base = pl.multiple_of(wid * EPT, 8)
    pltpu.sync_copy(src_hbm.at[pl.ds(base, EPT)], srcs_v)
    pltpu.sync_copy(dst_hbm.at[wid], dsts_v)
    pltpu.sync_copy(zeros_hbm, acc_sh.at[pl.ds(row0, RPT)])
    plsc.subcore_barrier()

    def src_at(i):
        return srcs_v.at[pl.ds(pl.multiple_of(i * K, 8), K)]

    pltpu.async_copy(hp_hbm.at[src_at(0)], rows0_v, sem0)

    def body(q, c):
        i0 = 2 * q
        i1 = i0 + 1
        i2 = i0 + 2
        pltpu.async_copy(hp_hbm.at[src_at(i1)], rows1_v, sem1)
        pltpu.make_async_copy(hp_hbm.at[src_at(i0)], rows0_v, sem0).wait()
        pltpu.sync_copy(rows0_v, acc_sh.at[dsts_v.at[i0]], add=True)
        pltpu.async_copy(hp_hbm.at[src_at(i2)], rows0_v, sem0)
        pltpu.make_async_copy(hp_hbm.at[src_at(i1)], rows1_v, sem1).wait()
        pltpu.sync_copy(rows1_v, acc_sh.at[dsts_v.at[i1]], add=True)
        return c

    lax.fori_loop(0, (CHUNKS - 1) // 2, body, 0)
    last = CHUNKS - 1
    pltpu.make_async_copy(hp_hbm.at[src_at(last)], rows0_v, sem0).wait()
    pltpu.sync_copy(rows0_v, acc_sh.at[dsts_v.at[last]], add=True)
    plsc.subcore_barrier()
    out_row0 = pl.multiple_of(cid * NP + sid * RPT, 8)
    pltpu.sync_copy(acc_sh.at[pl.ds(row0, RPT)], out_hbm.at[pl.ds(out_row0, RPT)])


def _dis_from_parts(parts_ref):
    ones = jnp.ones((NW, 1), jnp.float32)
    deg = lax.dot_general(parts_ref[...], ones, (((0,), (0,)), ((), ())),
                          preferred_element_type=jnp.float32)
    return lax.rsqrt(deg[0:N_NODES] + 1.0)


def _tc_prescale(x_ref, w_ref, parts_ref, out_ref):
    dis = _dis_from_parts(parts_ref)
    h = jnp.dot(x_ref[...], w_ref[...], preferred_element_type=jnp.float32)
    out_ref[...] = h * dis


def _tc_mid(s_ref, hp_ref, parts_ref, b_ref, w_ref, out_ref):
    dis = _dis_from_parts(parts_ref)
    s = s_ref[0:N_NODES] + s_ref[NP:NP + N_NODES] + hp_ref[...]
    h = jnp.maximum(dis * s + b_ref[...], 0.0)
    out_ref[...] = jnp.dot(h, w_ref[...], preferred_element_type=jnp.float32) * dis


def _tc_head(s_ref, hp_ref, parts_ref, b_ref, batch_ref, wl_ref, bl_ref, out_ref):
    dis = _dis_from_parts(parts_ref)
    s = s_ref[0:N_NODES] + s_ref[NP:NP + N_NODES] + hp_ref[...]
    h = jnp.maximum(dis * s + b_ref[...], 0.0)
    gid = lax.broadcasted_iota(jnp.int32, (G, N_NODES), 0)
    p = (gid == batch_ref[...]).astype(jnp.float32)
    sums = jnp.dot(p, h, preferred_element_type=jnp.float32)
    counts = jnp.sum(p, axis=1, keepdims=True)
    g = sums / jnp.maximum(counts, 1.0)
    logit = jnp.dot(g, wl_ref[...], preferred_element_type=jnp.float32) + bl_ref[...]
    out_ref[...] = 1.0 / (1.0 + jnp.exp(-logit))


def kernel(x, edge_index, batch, W1, b1, W2, b2, Wl, bl):
    src = edge_index[0].astype(jnp.int32)
    dst = edge_index[1].astype(jnp.int32)
    dst3d = dst.reshape(NW, CHUNKS, K)
    batch2d = batch.astype(jnp.int32).reshape(1, N_NODES)
    zeros_cnt = jnp.zeros((NP,), jnp.float32)
    zeros_feat = jnp.zeros((RPT, D), jnp.float32)
    f32 = jnp.float32

    parts = _sc_degree(dst, zeros_cnt).reshape(NW, NP)

    h1p = pl.pallas_call(
        _tc_prescale, out_shape=jax.ShapeDtypeStruct((N_NODES, D), f32),
    )(x, W1, parts)

    s1 = _sc_scatter(h1p, src, dst3d, zeros_feat)

    h2p = pl.pallas_call(
        _tc_mid, out_shape=jax.ShapeDtypeStruct((N_NODES, D), f32),
    )(s1, h1p, parts, b1.reshape(1, D), W2)

    s2 = _sc_scatter(h2p, src, dst3d, zeros_feat)

    out = pl.pallas_call(
        _tc_head, out_shape=jax.ShapeDtypeStruct((G, 1), f32),
    )(s2, h2p, parts, b2.reshape(1, D), batch2d, Wl, bl.reshape(1, 1))
    return out.reshape(-1)
